# baseline (speedup 1.0000x reference)
import functools

import jax
import jax.numpy as jnp
from jax import lax
from jax.experimental import pallas as pl
from jax.experimental.pallas import tpu as pltpu
from jax.experimental.pallas import tpu_sc as plsc

_N = 10000
_E = 320000
_D = 128

_NC = 2
_NS = 16
_NW = _NC * _NS
_K = 128
_CHUNKS = 80
_EPT = _K * _CHUNKS
_EPAD = _EPT * _NW
_NP = 10240
_RPT = _NP // _NS
_ZR = 16
_HC = _CHUNKS // 2

_sc_mesh = plsc.VectorSubcoreMesh(core_axis_name="c", subcore_axis_name="s")
_sc_params = pltpu.CompilerParams(use_tc_tiling_on_sc=False)


@functools.partial(
    pl.kernel,
    out_type=jax.ShapeDtypeStruct((_NC * _NP,), jnp.float32),
    mesh=_sc_mesh,
    compiler_params=_sc_params,
    scratch_types=[
        pltpu.VMEM((_CHUNKS, _K), jnp.int32),
        pltpu.VMEM((_K,), jnp.float32),
        pltpu.VMEM((_RPT,), jnp.float32),
        pltpu.VMEM_SHARED((_NP,), jnp.float32),
        pltpu.SemaphoreType.DMA,
    ],
)
def _sc_degree(dst_hbm, ones_hbm, zcol_hbm, out_hbm, idx_d, ones_v, zcol_v, acc_sh, sem):
    c = lax.axis_index("c")
    s = lax.axis_index("s")
    wid = s * _NC + c
    idx_load = pltpu.make_async_copy(dst_hbm.at[wid], idx_d, sem)
    idx_load.start()
    pltpu.sync_copy(ones_hbm, ones_v)
    pltpu.sync_copy(zcol_hbm, zcol_v)
    pltpu.sync_copy(zcol_v, acc_sh.at[pl.ds(s * _RPT, _RPT)])
    idx_load.wait()
    plsc.subcore_barrier()

    def body(j, carry):
        pltpu.sync_copy(ones_v, acc_sh.at[idx_d.at[j]], add=True)
        return carry

    lax.fori_loop(0, _CHUNKS, body, 0)
    plsc.subcore_barrier()
    pltpu.sync_copy(acc_sh.at[pl.ds(s * _RPT, _RPT)],
                    out_hbm.at[pl.ds(c * _NP + s * _RPT, _RPT)])


@functools.partial(
    pl.kernel,
    out_type=jax.ShapeDtypeStruct((_NC, _NP, _D), jnp.float32),
    mesh=_sc_mesh,
    compiler_params=_sc_params,
    scratch_types=[
        pltpu.VMEM((_HC, _K), jnp.int32),
        pltpu.VMEM((_HC, _K), jnp.int32),
        pltpu.VMEM((_K, _D), jnp.float32),
        pltpu.VMEM((_K, _D), jnp.float32),
        pltpu.VMEM((_ZR, _D), jnp.float32),
        pltpu.VMEM_SHARED((_NP, _D), jnp.float32),
        pltpu.SemaphoreType.DMA,
        pltpu.SemaphoreType.DMA,
        pltpu.SemaphoreType.DMA,
        pltpu.SemaphoreType.DMA,
        pltpu.SemaphoreType.DMA,
    ],
)
def _sc_scatter(g_hbm, src_hbm, dst_hbm, zrows_hbm, out_hbm,
                idx_s, idx_d, rows0, rows1, zbuf_v, acc_sh,
                sem_i, sem_g0, sem_g1, sem_s0, sem_s1):
    c = lax.axis_index("c")
    s = lax.axis_index("s")
    wid = s * _NC + c

    def load_idx(h):
        ls = pltpu.make_async_copy(src_hbm.at[wid, pl.ds(h * _HC, _HC)], idx_s, sem_i)
        ld = pltpu.make_async_copy(dst_hbm.at[wid, pl.ds(h * _HC, _HC)], idx_d, sem_i)
        ls.start()
        ld.start()
        return ls, ld

    ls, ld = load_idx(0)
    pltpu.sync_copy(zrows_hbm, zbuf_v)

    def zbody(j, carry):
        pltpu.sync_copy(zbuf_v, acc_sh.at[pl.ds(s * _RPT + j * _ZR, _ZR)])
        return carry

    lax.fori_loop(0, _RPT // _ZR, zbody, 0)
    ls.wait()
    ld.wait()
    plsc.subcore_barrier()

    def gather(j, buf, sem):
        return pltpu.make_async_copy(g_hbm.at[idx_s.at[j]], buf, sem)

    def scatter(j, buf, sem):
        return pltpu.async_copy(buf, acc_sh.at[idx_d.at[j]], sem, add=True)

    for h in range(2):
        gather(0, rows0, sem_g0).start()
        gather(1, rows1, sem_g1).start()

        def body(i, carry):
            j0 = 2 * i
            j1 = 2 * i + 1
            gather(j0, rows0, sem_g0).wait()
            sc0 = scatter(j0, rows0, sem_s0)
            gather(j1, rows1, sem_g1).wait()
            sc1 = scatter(j1, rows1, sem_s1)
            sc0.wait()

            @pl.when(i < _HC // 2 - 1)
            def _():
                gather(j0 + 2, rows0, sem_g0).start()

            sc1.wait()

            @pl.when(i < _HC // 2 - 1)
            def _():
                gather(j1 + 2, rows1, sem_g1).start()

            return carry

        lax.fori_loop(0, _HC // 2, body, 0)
        if h == 0:
            ls, ld = load_idx(1)
            ls.wait()
            ld.wait()
    plsc.subcore_barrier()
    pltpu.sync_copy(acc_sh.at[pl.ds(s * _RPT, _RPT)],
                    out_hbm.at[c, pl.ds(s * _RPT, _RPT)])


def _tc1_body(x_ref, w1_ref, dis_ref, g_ref):
    h = jnp.dot(x_ref[...], w1_ref[...], preferred_element_type=jnp.float32)
    g_ref[...] = dis_ref[...] * h


def _tc2_body(p_ref, g1_ref, dis_ref, b1_ref, w2_ref, g2_ref):
    ssum = p_ref[0, :_N, :] + p_ref[1, :_N, :]
    dis = dis_ref[...]
    h1 = dis * (ssum + g1_ref[...]) + b1_ref[...]
    a = jax.nn.gelu(h1)
    h2 = jnp.dot(a, w2_ref[...], preferred_element_type=jnp.float32)
    g2_ref[...] = dis * h2


def _tc3_body(p_ref, g2_ref, dis_ref, b2_ref, out_ref):
    ssum = p_ref[0, :_N, :] + p_ref[1, :_N, :]
    out_ref[...] = dis_ref[...] * (ssum + g2_ref[...]) + b2_ref[...]


def kernel(x, edge_index, W1, b1, W2, b2):
    src = edge_index[0].astype(jnp.int32)
    dst = edge_index[1].astype(jnp.int32)
    npad = _EPAD - _E
    pad_i = jnp.arange(npad, dtype=jnp.int32)
    src_p = jnp.concatenate([src, pad_i]).reshape(_NW, _CHUNKS, _K)
    dst_p = jnp.concatenate([dst, _N + pad_i % (_NP - _N)]).reshape(_NW, _CHUNKS, _K)
    ones_k = jnp.ones((_K,), jnp.float32)
    zcol = jnp.zeros((_RPT,), jnp.float32)
    zrows = jnp.zeros((_ZR, _D), jnp.float32)

    deg2 = _sc_degree(dst_p, ones_k, zcol)
    deg = deg2[: _N] + deg2[_NP : _NP + _N] + 1.0
    dis = lax.rsqrt(deg)[:, None]

    g1 = pl.pallas_call(
        _tc1_body,
        out_shape=jax.ShapeDtypeStruct((_N, _D), jnp.float32),
    )(x, W1, dis)

    p1 = _sc_scatter(g1, src_p, dst_p, zrows)

    g2 = pl.pallas_call(
        _tc2_body,
        out_shape=jax.ShapeDtypeStruct((_N, _D), jnp.float32),
    )(p1, g1, dis, b1.reshape(1, _D), W2)

    p2 = _sc_scatter(g2, src_p, dst_p, zrows)

    out = pl.pallas_call(
        _tc3_body,
        out_shape=jax.ShapeDtypeStruct((_N, _D), jnp.float32),
    )(p2, g2, dis, b2.reshape(1, _D))

    return out


# speedup vs baseline: 1.0775x; 1.0775x over previous
import functools

import jax
import jax.numpy as jnp
from jax import lax
from jax.experimental import pallas as pl
from jax.experimental.pallas import tpu as pltpu
from jax.experimental.pallas import tpu_sc as plsc

_N = 10000
_E = 320000
_D = 128

_NC = 2
_NS = 16
_NW = _NC * _NS
_K = 128
_CHUNKS = 80
_EPT = _K * _CHUNKS
_EPAD = _EPT * _NW
_NP = 10240
_RPT = _NP // _NS
_ZR = 16
_HC = _CHUNKS // 2

_sc_mesh = plsc.VectorSubcoreMesh(core_axis_name="c", subcore_axis_name="s")
_sc_params = pltpu.CompilerParams(use_tc_tiling_on_sc=False)


@functools.partial(
    pl.kernel,
    out_type=jax.ShapeDtypeStruct((_NC * _NP,), jnp.float32),
    mesh=_sc_mesh,
    compiler_params=_sc_params,
    scratch_types=[
        pltpu.VMEM((_CHUNKS, _K), jnp.int32),
        pltpu.VMEM((_K,), jnp.float32),
        pltpu.VMEM((_RPT,), jnp.float32),
        pltpu.VMEM_SHARED((_NP,), jnp.float32),
        pltpu.SemaphoreType.DMA,
    ],
)
def _sc_degree(dst_hbm, ones_hbm, zcol_hbm, out_hbm, idx_d, ones_v, zcol_v, acc_sh, sem):
    c = lax.axis_index("c")
    s = lax.axis_index("s")
    wid = s * _NC + c
    idx_load = pltpu.make_async_copy(dst_hbm.at[wid], idx_d, sem)
    idx_load.start()
    pltpu.sync_copy(ones_hbm, ones_v)
    pltpu.sync_copy(zcol_hbm, zcol_v)
    pltpu.sync_copy(zcol_v, acc_sh.at[pl.ds(s * _RPT, _RPT)])
    idx_load.wait()
    plsc.subcore_barrier()

    def body(j, carry):
        pltpu.sync_copy(ones_v, acc_sh.at[idx_d.at[j]], add=True)
        return carry

    lax.fori_loop(0, _CHUNKS, body, 0)
    plsc.subcore_barrier()
    pltpu.sync_copy(acc_sh.at[pl.ds(s * _RPT, _RPT)],
                    out_hbm.at[pl.ds(c * _NP + s * _RPT, _RPT)])


@functools.partial(
    pl.kernel,
    out_type=jax.ShapeDtypeStruct((_NC, _NP, _D), jnp.float32),
    mesh=_sc_mesh,
    compiler_params=_sc_params,
    scratch_types=[
        pltpu.VMEM((_HC, _K), jnp.int32),
        pltpu.VMEM((_HC, _K), jnp.int32),
        pltpu.VMEM((_K, _D), jnp.float32),
        pltpu.VMEM((_K, _D), jnp.float32),
        pltpu.VMEM((_ZR, _D), jnp.float32),
        pltpu.VMEM_SHARED((_NP, _D), jnp.float32),
        pltpu.SemaphoreType.DMA,
        pltpu.SemaphoreType.DMA,
        pltpu.SemaphoreType.DMA,
        pltpu.SemaphoreType.DMA,
        pltpu.SemaphoreType.DMA,
    ],
)
def _sc_scatter(g_hbm, src_hbm, dst_hbm, zrows_hbm, out_hbm,
                idx_s, idx_d, rows0, rows1, zbuf_v, acc_sh,
                sem_i, sem_g0, sem_g1, sem_s0, sem_s1):
    c = lax.axis_index("c")
    s = lax.axis_index("s")
    wid = s * _NC + c

    def load_idx(h):
        ls = pltpu.make_async_copy(src_hbm.at[wid, pl.ds(h * _HC, _HC)], idx_s, sem_i)
        ld = pltpu.make_async_copy(dst_hbm.at[wid, pl.ds(h * _HC, _HC)], idx_d, sem_i)
        ls.start()
        ld.start()
        return ls, ld

    ls, ld = load_idx(0)
    pltpu.sync_copy(zrows_hbm, zbuf_v)

    def zbody(j, carry):
        pltpu.sync_copy(zbuf_v, acc_sh.at[pl.ds(s * _RPT + j * _ZR, _ZR)])
        return carry

    lax.fori_loop(0, _RPT // _ZR, zbody, 0)
    ls.wait()
    ld.wait()
    plsc.subcore_barrier()

    def gather(j, buf, sem):
        return pltpu.make_async_copy(g_hbm.at[idx_s.at[j]], buf, sem)

    for h in range(2):
        gather(0, rows0, sem_g0).start()

        def body(i, carry):
            j0 = 2 * i
            j1 = 2 * i + 1
            gather(j0, rows0, sem_g0).wait()
            gather(j1, rows1, sem_g1).start()
            pltpu.sync_copy(rows0, acc_sh.at[idx_d.at[j0]], add=True)
            gather(j1, rows1, sem_g1).wait()

            @pl.when(i < _HC // 2 - 1)
            def _():
                gather(j0 + 2, rows0, sem_g0).start()

            pltpu.sync_copy(rows1, acc_sh.at[idx_d.at[j1]], add=True)
            return carry

        lax.fori_loop(0, _HC // 2, body, 0)
        if h == 0:
            ls, ld = load_idx(1)
            ls.wait()
            ld.wait()
    plsc.subcore_barrier()
    pltpu.sync_copy(acc_sh.at[pl.ds(s * _RPT, _RPT)],
                    out_hbm.at[c, pl.ds(s * _RPT, _RPT)])


def _tca_body(x_ref, w1_ref, h_ref):
    h_ref[...] = jnp.dot(x_ref[...], w1_ref[...],
                         preferred_element_type=jnp.float32)


def _tcb_body(h_ref, dis_ref, g_ref):
    g_ref[...] = dis_ref[...] * h_ref[...]


def _tc2_body(p_ref, g1_ref, dis_ref, b1_ref, w2_ref, g2_ref):
    ssum = p_ref[0, :_N, :] + p_ref[1, :_N, :]
    dis = dis_ref[...]
    h1 = dis * (ssum + g1_ref[...]) + b1_ref[...]
    a = jax.nn.gelu(h1)
    h2 = jnp.dot(a, w2_ref[...], preferred_element_type=jnp.float32)
    g2_ref[...] = dis * h2


def _tc3_body(p_ref, g2_ref, dis_ref, b2_ref, out_ref):
    ssum = p_ref[0, :_N, :] + p_ref[1, :_N, :]
    out_ref[...] = dis_ref[...] * (ssum + g2_ref[...]) + b2_ref[...]


def kernel(x, edge_index, W1, b1, W2, b2):
    src = edge_index[0].astype(jnp.int32)
    dst = edge_index[1].astype(jnp.int32)
    npad = _EPAD - _E
    pad_i = jnp.arange(npad, dtype=jnp.int32)
    src_p = jnp.concatenate([src, pad_i]).reshape(_NW, _CHUNKS, _K)
    dst_p = jnp.concatenate([dst, _N + pad_i % (_NP - _N)]).reshape(_NW, _CHUNKS, _K)
    ones_k = jnp.ones((_K,), jnp.float32)
    zcol = jnp.zeros((_RPT,), jnp.float32)
    zrows = jnp.zeros((_ZR, _D), jnp.float32)

    deg2 = _sc_degree(dst_p, ones_k, zcol)
    h1 = pl.pallas_call(
        _tca_body,
        out_shape=jax.ShapeDtypeStruct((_N, _D), jnp.float32),
    )(x, W1)
    deg = deg2[: _N] + deg2[_NP : _NP + _N] + 1.0
    dis = lax.rsqrt(deg)[:, None]

    g1 = pl.pallas_call(
        _tcb_body,
        out_shape=jax.ShapeDtypeStruct((_N, _D), jnp.float32),
    )(h1, dis)

    p1 = _sc_scatter(g1, src_p, dst_p, zrows)

    g2 = pl.pallas_call(
        _tc2_body,
        out_shape=jax.ShapeDtypeStruct((_N, _D), jnp.float32),
    )(p1, g1, dis, b1.reshape(1, _D), W2)

    p2 = _sc_scatter(g2, src_p, dst_p, zrows)

    out = pl.pallas_call(
        _tc3_body,
        out_shape=jax.ShapeDtypeStruct((_N, _D), jnp.float32),
    )(p2, g2, dis, b2.reshape(1, _D))

    return out


# speedup vs baseline: 1.0857x; 1.0076x over previous
import functools

import jax
import jax.numpy as jnp
from jax import lax
from jax.experimental import pallas as pl
from jax.experimental.pallas import tpu as pltpu
from jax.experimental.pallas import tpu_sc as plsc

_N = 10000
_E = 320000
_D = 128

_NC = 2
_NS = 16
_NW = _NC * _NS
_K = 128
_CHUNKS = 80
_EPT = _K * _CHUNKS
_EPAD = _EPT * _NW
_NP = 10240
_RPT = _NP // _NS
_ZR = 16
_HC = _CHUNKS // 2

_sc_mesh = plsc.VectorSubcoreMesh(core_axis_name="c", subcore_axis_name="s")
_sc_params = pltpu.CompilerParams(use_tc_tiling_on_sc=False)


@functools.partial(
    pl.kernel,
    out_type=jax.ShapeDtypeStruct((_NC * _NP,), jnp.float32),
    mesh=_sc_mesh,
    compiler_params=_sc_params,
    scratch_types=[
        pltpu.VMEM((_CHUNKS, _K), jnp.int32),
        pltpu.VMEM((_K,), jnp.float32),
        pltpu.VMEM((_RPT,), jnp.float32),
        pltpu.VMEM_SHARED((_NP,), jnp.float32),
        pltpu.SemaphoreType.DMA,
    ],
)
def _sc_degree(dst_hbm, ones_hbm, zcol_hbm, out_hbm, idx_d, ones_v, zcol_v, acc_sh, sem):
    c = lax.axis_index("c")
    s = lax.axis_index("s")
    wid = s * _NC + c
    idx_load = pltpu.make_async_copy(dst_hbm.at[wid], idx_d, sem)
    idx_load.start()
    pltpu.sync_copy(ones_hbm, ones_v)
    pltpu.sync_copy(zcol_hbm, zcol_v)
    pltpu.sync_copy(zcol_v, acc_sh.at[pl.ds(s * _RPT, _RPT)])
    idx_load.wait()
    plsc.subcore_barrier()

    def body(j, carry):
        pltpu.sync_copy(ones_v, acc_sh.at[idx_d.at[j]], add=True)
        return carry

    lax.fori_loop(0, _CHUNKS, body, 0)
    plsc.subcore_barrier()
    pltpu.sync_copy(acc_sh.at[pl.ds(s * _RPT, _RPT)],
                    out_hbm.at[pl.ds(c * _NP + s * _RPT, _RPT)])


@functools.partial(
    pl.kernel,
    out_type=jax.ShapeDtypeStruct((_NC, _NP, _D), jnp.float32),
    mesh=_sc_mesh,
    compiler_params=_sc_params,
    scratch_types=[
        pltpu.VMEM((_HC, _K), jnp.int32),
        pltpu.VMEM((_HC, _K), jnp.int32),
        pltpu.VMEM((_K, _D), jnp.float32),
        pltpu.VMEM((_K, _D), jnp.float32),
        pltpu.VMEM((_ZR, _D), jnp.float32),
        pltpu.VMEM_SHARED((_NP, _D), jnp.float32),
        pltpu.SemaphoreType.DMA,
        pltpu.SemaphoreType.DMA,
        pltpu.SemaphoreType.DMA,
        pltpu.SemaphoreType.DMA,
        pltpu.SemaphoreType.DMA,
    ],
)
def _sc_scatter(g_hbm, src_hbm, dst_hbm, zrows_hbm, out_hbm,
                idx_s, idx_d, rows0, rows1, zbuf_v, acc_sh,
                sem_i, sem_g0, sem_g1, sem_s0, sem_s1):
    c = lax.axis_index("c")
    s = lax.axis_index("s")
    wid = s * _NC + c

    def load_idx(h):
        ls = pltpu.make_async_copy(src_hbm.at[wid, pl.ds(h * _HC, _HC)], idx_s, sem_i)
        ld = pltpu.make_async_copy(dst_hbm.at[wid, pl.ds(h * _HC, _HC)], idx_d, sem_i)
        ls.start()
        ld.start()
        return ls, ld

    ls, ld = load_idx(0)
    pltpu.sync_copy(zrows_hbm, zbuf_v)

    def zbody(j, carry):
        pltpu.sync_copy(zbuf_v, acc_sh.at[pl.ds(s * _RPT + j * _ZR, _ZR)])
        return carry

    lax.fori_loop(0, _RPT // _ZR, zbody, 0)
    ls.wait()
    ld.wait()
    plsc.subcore_barrier()

    _KH = _K // 2

    def gather_half(j, half, buf, sem):
        return pltpu.make_async_copy(
            g_hbm.at[idx_s.at[j, pl.ds(half * _KH, _KH)]],
            buf.at[pl.ds(half * _KH, _KH)], sem)

    def gather_start(j, buf, sem):
        gather_half(j, 0, buf, sem).start()
        gather_half(j, 1, buf, sem).start()

    def gather_wait(j, buf, sem):
        gather_half(j, 0, buf, sem).wait()
        gather_half(j, 1, buf, sem).wait()

    for h in range(2):
        gather_start(0, rows0, sem_g0)

        def body(i, carry):
            j0 = 2 * i
            j1 = 2 * i + 1
            gather_wait(j0, rows0, sem_g0)
            gather_start(j1, rows1, sem_g1)
            pltpu.sync_copy(rows0, acc_sh.at[idx_d.at[j0]], add=True)
            gather_wait(j1, rows1, sem_g1)

            @pl.when(i < _HC // 2 - 1)
            def _():
                gather_start(j0 + 2, rows0, sem_g0)

            pltpu.sync_copy(rows1, acc_sh.at[idx_d.at[j1]], add=True)
            return carry

        lax.fori_loop(0, _HC // 2, body, 0)
        if h == 0:
            ls, ld = load_idx(1)
            ls.wait()
            ld.wait()
    plsc.subcore_barrier()
    pltpu.sync_copy(acc_sh.at[pl.ds(s * _RPT, _RPT)],
                    out_hbm.at[c, pl.ds(s * _RPT, _RPT)])


def _tc1_body(x_ref, w1_ref, dis_ref, g_ref):
    h = jnp.dot(x_ref[...], w1_ref[...], preferred_element_type=jnp.float32)
    g_ref[...] = dis_ref[...] * h


def _tc2_body(p_ref, g1_ref, dis_ref, b1_ref, w2_ref, g2_ref):
    ssum = p_ref[0, :_N, :] + p_ref[1, :_N, :]
    dis = dis_ref[...]
    h1 = dis * (ssum + g1_ref[...]) + b1_ref[...]
    a = jax.nn.gelu(h1)
    h2 = jnp.dot(a, w2_ref[...], preferred_element_type=jnp.float32)
    g2_ref[...] = dis * h2


def _tc3_body(p_ref, g2_ref, dis_ref, b2_ref, out_ref):
    ssum = p_ref[0, :_N, :] + p_ref[1, :_N, :]
    out_ref[...] = dis_ref[...] * (ssum + g2_ref[...]) + b2_ref[...]


def kernel(x, edge_index, W1, b1, W2, b2):
    src = edge_index[0].astype(jnp.int32)
    dst = edge_index[1].astype(jnp.int32)
    npad = _EPAD - _E
    pad_i = jnp.arange(npad, dtype=jnp.int32)
    src_p = jnp.concatenate([src, pad_i]).reshape(_NW, _CHUNKS, _K)
    dst_p = jnp.concatenate([dst, _N + pad_i % (_NP - _N)]).reshape(_NW, _CHUNKS, _K)
    ones_k = jnp.ones((_K,), jnp.float32)
    zcol = jnp.zeros((_RPT,), jnp.float32)
    zrows = jnp.zeros((_ZR, _D), jnp.float32)

    deg2 = _sc_degree(dst_p, ones_k, zcol)
    deg = deg2[: _N] + deg2[_NP : _NP + _N] + 1.0
    dis = lax.rsqrt(deg)[:, None]

    g1 = pl.pallas_call(
        _tc1_body,
        out_shape=jax.ShapeDtypeStruct((_N, _D), jnp.float32),
    )(x, W1, dis)

    p1 = _sc_scatter(g1, src_p, dst_p, zrows)

    g2 = pl.pallas_call(
        _tc2_body,
        out_shape=jax.ShapeDtypeStruct((_N, _D), jnp.float32),
    )(p1, g1, dis, b1.reshape(1, _D), W2)

    p2 = _sc_scatter(g2, src_p, dst_p, zrows)

    out = pl.pallas_call(
        _tc3_body,
        out_shape=jax.ShapeDtypeStruct((_N, _D), jnp.float32),
    )(p2, g2, dis, b2.reshape(1, _D))

    return out


# speedup vs baseline: 1.0960x; 1.0095x over previous
import functools

import jax
import jax.numpy as jnp
from jax import lax
from jax.experimental import pallas as pl
from jax.experimental.pallas import tpu as pltpu
from jax.experimental.pallas import tpu_sc as plsc

_N = 10000
_E = 320000
_D = 128

_NC = 2
_NS = 16
_NW = _NC * _NS
_K = 128
_CHUNKS = 80
_EPT = _K * _CHUNKS
_EPAD = _EPT * _NW
_NP = 10240
_RPT = _NP // _NS
_ZR = 16
_HC = _CHUNKS // 2

_sc_mesh = plsc.VectorSubcoreMesh(core_axis_name="c", subcore_axis_name="s")
_sc_params = pltpu.CompilerParams(use_tc_tiling_on_sc=False)


@functools.partial(
    pl.kernel,
    out_type=jax.ShapeDtypeStruct((_NC * _NP,), jnp.float32),
    mesh=_sc_mesh,
    compiler_params=_sc_params,
    scratch_types=[
        pltpu.VMEM((_CHUNKS, _K), jnp.int32),
        pltpu.VMEM((_K,), jnp.float32),
        pltpu.VMEM((_RPT,), jnp.float32),
        pltpu.VMEM_SHARED((_NP,), jnp.float32),
        pltpu.SemaphoreType.DMA,
    ],
)
def _sc_degree(dst_hbm, ones_hbm, zcol_hbm, out_hbm, idx_d, ones_v, zcol_v, acc_sh, sem):
    c = lax.axis_index("c")
    s = lax.axis_index("s")
    wid = s * _NC + c
    idx_load = pltpu.make_async_copy(dst_hbm.at[wid], idx_d, sem)
    idx_load.start()
    pltpu.sync_copy(ones_hbm, ones_v)
    pltpu.sync_copy(zcol_hbm, zcol_v)
    pltpu.sync_copy(zcol_v, acc_sh.at[pl.ds(s * _RPT, _RPT)])
    idx_load.wait()
    plsc.subcore_barrier()

    def body(j, carry):
        pltpu.sync_copy(ones_v, acc_sh.at[idx_d.at[j]], add=True)
        return carry

    lax.fori_loop(0, _CHUNKS, body, 0)
    plsc.subcore_barrier()
    pltpu.sync_copy(acc_sh.at[pl.ds(s * _RPT, _RPT)],
                    out_hbm.at[pl.ds(c * _NP + s * _RPT, _RPT)])


@functools.partial(
    pl.kernel,
    out_type=jax.ShapeDtypeStruct((_NC, _NP, _D), jnp.float32),
    mesh=_sc_mesh,
    compiler_params=_sc_params,
    scratch_types=[
        pltpu.VMEM((_HC, _K), jnp.int32),
        pltpu.VMEM((_HC, _K), jnp.int32),
        pltpu.VMEM((_K, _D), jnp.float32),
        pltpu.VMEM((_K, _D), jnp.float32),
        pltpu.VMEM((_ZR, _D), jnp.float32),
        pltpu.VMEM_SHARED((_NP, _D), jnp.float32),
        pltpu.SemaphoreType.DMA,
        pltpu.SemaphoreType.DMA,
        pltpu.SemaphoreType.DMA,
        pltpu.SemaphoreType.DMA,
        pltpu.SemaphoreType.DMA,
    ],
)
def _sc_scatter(g_hbm, src_hbm, dst_hbm, zrows_hbm, out_hbm,
                idx_s, idx_d, rows0, rows1, zbuf_v, acc_sh,
                sem_i, sem_g0, sem_g1, sem_s0, sem_s1):
    c = lax.axis_index("c")
    s = lax.axis_index("s")
    wid = s * _NC + c

    def load_idx(h):
        ls = pltpu.make_async_copy(src_hbm.at[wid, pl.ds(h * _HC, _HC)], idx_s, sem_i)
        ld = pltpu.make_async_copy(dst_hbm.at[wid, pl.ds(h * _HC, _HC)], idx_d, sem_i)
        ls.start()
        ld.start()
        return ls, ld

    ls, ld = load_idx(0)
    pltpu.sync_copy(zrows_hbm, zbuf_v)

    def zcopy(j):
        return pltpu.make_async_copy(
            zbuf_v, acc_sh.at[pl.ds(s * _RPT + j * _ZR, _ZR)], sem_g0)

    def zstart(j, carry):
        zcopy(j).start()
        return carry

    def zdrain(j, carry):
        zcopy(j).wait()
        return carry

    lax.fori_loop(0, _RPT // _ZR, zstart, 0)
    lax.fori_loop(0, _RPT // _ZR, zdrain, 0)
    ls.wait()
    ld.wait()
    plsc.subcore_barrier()

    _KH = _K // 2

    def gather_half(j, half, buf, sem):
        return pltpu.make_async_copy(
            g_hbm.at[idx_s.at[j, pl.ds(half * _KH, _KH)]],
            buf.at[pl.ds(half * _KH, _KH)], sem)

    def gather_start(j, buf, sem):
        gather_half(j, 0, buf, sem).start()
        gather_half(j, 1, buf, sem).start()

    def gather_wait(j, buf, sem):
        gather_half(j, 0, buf, sem).wait()
        gather_half(j, 1, buf, sem).wait()

    for h in range(2):
        gather_start(0, rows0, sem_g0)

        def body(i, carry):
            j0 = 2 * i
            j1 = 2 * i + 1
            gather_wait(j0, rows0, sem_g0)
            gather_start(j1, rows1, sem_g1)
            pltpu.sync_copy(rows0, acc_sh.at[idx_d.at[j0]], add=True)
            gather_wait(j1, rows1, sem_g1)

            @pl.when(i < _HC // 2 - 1)
            def _():
                gather_start(j0 + 2, rows0, sem_g0)

            pltpu.sync_copy(rows1, acc_sh.at[idx_d.at[j1]], add=True)
            return carry

        lax.fori_loop(0, _HC // 2, body, 0)
        if h == 0:
            ls, ld = load_idx(1)
            ls.wait()
            ld.wait()
    plsc.subcore_barrier()
    pltpu.sync_copy(acc_sh.at[pl.ds(s * _RPT, _RPT)],
                    out_hbm.at[c, pl.ds(s * _RPT, _RPT)])


def _tc1_body(x_ref, w1_ref, dis_ref, g_ref):
    h = jnp.dot(x_ref[...], w1_ref[...], preferred_element_type=jnp.float32)
    g_ref[...] = dis_ref[...] * h


def _tc2_body(p_ref, g1_ref, dis_ref, b1_ref, w2_ref, g2_ref):
    ssum = p_ref[0, :_N, :] + p_ref[1, :_N, :]
    dis = dis_ref[...]
    h1 = dis * (ssum + g1_ref[...]) + b1_ref[...]
    a = jax.nn.gelu(h1)
    h2 = jnp.dot(a, w2_ref[...], preferred_element_type=jnp.float32)
    g2_ref[...] = dis * h2


def _tc3_body(p_ref, g2_ref, dis_ref, b2_ref, out_ref):
    ssum = p_ref[0, :_N, :] + p_ref[1, :_N, :]
    out_ref[...] = dis_ref[...] * (ssum + g2_ref[...]) + b2_ref[...]


def kernel(x, edge_index, W1, b1, W2, b2):
    src = edge_index[0].astype(jnp.int32)
    dst = edge_index[1].astype(jnp.int32)
    npad = _EPAD - _E
    pad_i = jnp.arange(npad, dtype=jnp.int32)
    src_p = jnp.concatenate([src, pad_i]).reshape(_NW, _CHUNKS, _K)
    dst_p = jnp.concatenate([dst, _N + pad_i % (_NP - _N)]).reshape(_NW, _CHUNKS, _K)
    ones_k = jnp.ones((_K,), jnp.float32)
    zcol = jnp.zeros((_RPT,), jnp.float32)
    zrows = jnp.zeros((_ZR, _D), jnp.float32)

    deg2 = _sc_degree(dst_p, ones_k, zcol)
    deg = deg2[: _N] + deg2[_NP : _NP + _N] + 1.0
    dis = lax.rsqrt(deg)[:, None]

    g1 = pl.pallas_call(
        _tc1_body,
        out_shape=jax.ShapeDtypeStruct((_N, _D), jnp.float32),
    )(x, W1, dis)

    p1 = _sc_scatter(g1, src_p, dst_p, zrows)

    g2 = pl.pallas_call(
        _tc2_body,
        out_shape=jax.ShapeDtypeStruct((_N, _D), jnp.float32),
    )(p1, g1, dis, b1.reshape(1, _D), W2)

    p2 = _sc_scatter(g2, src_p, dst_p, zrows)

    out = pl.pallas_call(
        _tc3_body,
        out_shape=jax.ShapeDtypeStruct((_N, _D), jnp.float32),
    )(p2, g2, dis, b2.reshape(1, _D))

    return out


# speedup vs baseline: 1.1172x; 1.0193x over previous
import functools

import jax
import jax.numpy as jnp
from jax import lax
from jax.experimental import pallas as pl
from jax.experimental.pallas import tpu as pltpu
from jax.experimental.pallas import tpu_sc as plsc

_N = 10000
_E = 320000
_D = 128

_NC = 2
_NS = 16
_NW = _NC * _NS
_K = 128
_CHUNKS = 80
_EPT = _K * _CHUNKS
_EPAD = _EPT * _NW
_NP = 10240
_RPT = _NP // _NS
_ZR = 16
_HC = _CHUNKS // 2

_sc_mesh = plsc.VectorSubcoreMesh(core_axis_name="c", subcore_axis_name="s")
_sc_params = pltpu.CompilerParams(use_tc_tiling_on_sc=False)


@functools.partial(
    pl.kernel,
    out_type=jax.ShapeDtypeStruct((_NC * _NP,), jnp.float32),
    mesh=_sc_mesh,
    compiler_params=_sc_params,
    scratch_types=[
        pltpu.VMEM((_CHUNKS, _K), jnp.int32),
        pltpu.VMEM((_K,), jnp.float32),
        pltpu.VMEM((_RPT,), jnp.float32),
        pltpu.VMEM_SHARED((_NP,), jnp.float32),
        pltpu.SemaphoreType.DMA,
        pltpu.SemaphoreType.DMA,
    ],
)
def _sc_degree(dst_hbm, ones_hbm, zcol_hbm, out_hbm, idx_d, ones_v, zcol_v,
               acc_sh, sem, sem_a):
    c = lax.axis_index("c")
    s = lax.axis_index("s")
    wid = s * _NC + c
    idx_load = pltpu.make_async_copy(dst_hbm.at[wid], idx_d, sem)
    idx_load.start()
    pltpu.sync_copy(ones_hbm, ones_v)
    pltpu.sync_copy(zcol_hbm, zcol_v)
    pltpu.sync_copy(zcol_v, acc_sh.at[pl.ds(s * _RPT, _RPT)])
    idx_load.wait()
    plsc.subcore_barrier()

    def astart(j, carry):
        pltpu.async_copy(ones_v, acc_sh.at[idx_d.at[j]], sem_a, add=True)
        return carry

    def adrain(j, carry):
        pltpu.make_async_copy(ones_v, acc_sh.at[idx_d.at[j]], sem_a).wait()
        return carry

    lax.fori_loop(0, _CHUNKS, astart, 0)
    lax.fori_loop(0, _CHUNKS, adrain, 0)
    plsc.subcore_barrier()
    pltpu.sync_copy(acc_sh.at[pl.ds(s * _RPT, _RPT)],
                    out_hbm.at[pl.ds(c * _NP + s * _RPT, _RPT)])


@functools.partial(
    pl.kernel,
    out_type=jax.ShapeDtypeStruct((_NC, _NP, _D), jnp.float32),
    mesh=_sc_mesh,
    compiler_params=_sc_params,
    scratch_types=[
        pltpu.VMEM((_HC, _K), jnp.int32),
        pltpu.VMEM((_HC, _K), jnp.int32),
        pltpu.VMEM((_K, _D), jnp.float32),
        pltpu.VMEM((_K, _D), jnp.float32),
        pltpu.VMEM((_ZR, _D), jnp.float32),
        pltpu.VMEM_SHARED((_NP, _D), jnp.float32),
        pltpu.SemaphoreType.DMA,
        pltpu.SemaphoreType.DMA,
        pltpu.SemaphoreType.DMA,
        pltpu.SemaphoreType.DMA,
        pltpu.SemaphoreType.DMA,
    ],
)
def _sc_scatter(g_hbm, src_hbm, dst_hbm, zrows_hbm, out_hbm,
                idx_s, idx_d, rows0, rows1, zbuf_v, acc_sh,
                sem_i, sem_g0, sem_g1, sem_s0, sem_s1):
    c = lax.axis_index("c")
    s = lax.axis_index("s")
    wid = s * _NC + c

    def load_idx(h):
        ls = pltpu.make_async_copy(src_hbm.at[wid, pl.ds(h * _HC, _HC)], idx_s, sem_i)
        ld = pltpu.make_async_copy(dst_hbm.at[wid, pl.ds(h * _HC, _HC)], idx_d, sem_i)
        ls.start()
        ld.start()
        return ls, ld

    ls, ld = load_idx(0)
    pltpu.sync_copy(zrows_hbm, zbuf_v)

    def zcopy(j):
        return pltpu.make_async_copy(
            zbuf_v, acc_sh.at[pl.ds(s * _RPT + j * _ZR, _ZR)], sem_g0)

    def zstart(j, carry):
        zcopy(j).start()
        return carry

    def zdrain(j, carry):
        zcopy(j).wait()
        return carry

    lax.fori_loop(0, _RPT // _ZR, zstart, 0)
    lax.fori_loop(0, _RPT // _ZR, zdrain, 0)
    ls.wait()
    ld.wait()
    plsc.subcore_barrier()

    _KH = _K // 2

    def gather_half(j, half, buf, sem):
        return pltpu.make_async_copy(
            g_hbm.at[idx_s.at[j, pl.ds(half * _KH, _KH)]],
            buf.at[pl.ds(half * _KH, _KH)], sem)

    def gather_start(j, buf, sem):
        gather_half(j, 0, buf, sem).start()
        gather_half(j, 1, buf, sem).start()

    def gather_wait(j, buf, sem):
        gather_half(j, 0, buf, sem).wait()
        gather_half(j, 1, buf, sem).wait()

    for h in range(2):
        gather_start(0, rows0, sem_g0)

        def body(i, carry):
            j0 = 2 * i
            j1 = 2 * i + 1
            gather_wait(j0, rows0, sem_g0)
            gather_start(j1, rows1, sem_g1)
            pltpu.sync_copy(rows0, acc_sh.at[idx_d.at[j0]], add=True)
            gather_wait(j1, rows1, sem_g1)

            @pl.when(i < _HC // 2 - 1)
            def _():
                gather_start(j0 + 2, rows0, sem_g0)

            pltpu.sync_copy(rows1, acc_sh.at[idx_d.at[j1]], add=True)
            return carry

        lax.fori_loop(0, _HC // 2, body, 0)
        if h == 0:
            ls, ld = load_idx(1)
            ls.wait()
            ld.wait()
    plsc.subcore_barrier()
    pltpu.sync_copy(acc_sh.at[pl.ds(s * _RPT, _RPT)],
                    out_hbm.at[c, pl.ds(s * _RPT, _RPT)])


def _tc1_body(x_ref, w1_ref, dis_ref, g_ref):
    h = jnp.dot(x_ref[...], w1_ref[...], preferred_element_type=jnp.float32)
    g_ref[...] = dis_ref[...] * h


def _tc2_body(p_ref, g1_ref, dis_ref, b1_ref, w2_ref, g2_ref):
    ssum = p_ref[0, :_N, :] + p_ref[1, :_N, :]
    dis = dis_ref[...]
    h1 = dis * (ssum + g1_ref[...]) + b1_ref[...]
    a = jax.nn.gelu(h1)
    h2 = jnp.dot(a, w2_ref[...], preferred_element_type=jnp.float32)
    g2_ref[...] = dis * h2


def _tc3_body(p_ref, g2_ref, dis_ref, b2_ref, out_ref):
    ssum = p_ref[0, :_N, :] + p_ref[1, :_N, :]
    out_ref[...] = dis_ref[...] * (ssum + g2_ref[...]) + b2_ref[...]


def kernel(x, edge_index, W1, b1, W2, b2):
    src = edge_index[0].astype(jnp.int32)
    dst = edge_index[1].astype(jnp.int32)
    npad = _EPAD - _E
    pad_i = jnp.arange(npad, dtype=jnp.int32)
    src_p = jnp.concatenate([src, pad_i]).reshape(_NW, _CHUNKS, _K)
    dst_p = jnp.concatenate([dst, _N + pad_i % (_NP - _N)]).reshape(_NW, _CHUNKS, _K)
    ones_k = jnp.ones((_K,), jnp.float32)
    zcol = jnp.zeros((_RPT,), jnp.float32)
    zrows = jnp.zeros((_ZR, _D), jnp.float32)

    deg2 = _sc_degree(dst_p, ones_k, zcol)
    deg = deg2[: _N] + deg2[_NP : _NP + _N] + 1.0
    dis = lax.rsqrt(deg)[:, None]

    g1 = pl.pallas_call(
        _tc1_body,
        out_shape=jax.ShapeDtypeStruct((_N, _D), jnp.float32),
    )(x, W1, dis)

    p1 = _sc_scatter(g1, src_p, dst_p, zrows)

    g2 = pl.pallas_call(
        _tc2_body,
        out_shape=jax.ShapeDtypeStruct((_N, _D), jnp.float32),
    )(p1, g1, dis, b1.reshape(1, _D), W2)

    p2 = _sc_scatter(g2, src_p, dst_p, zrows)

    out = pl.pallas_call(
        _tc3_body,
        out_shape=jax.ShapeDtypeStruct((_N, _D), jnp.float32),
    )(p2, g2, dis, b2.reshape(1, _D))

    return out
